# baseline (speedup 1.0000x reference)
import functools

import jax
import jax.numpy as jnp
from jax import lax
from jax.experimental import pallas as pl
from jax.experimental.pallas import tpu as pltpu
from jax.experimental.pallas import tpu_sc as plsc

N = 50000
D = 256
S = 128
NC = 2
NS = 16
L = 16
DC = D // NC
NV = DC // L
CB = 384
NB = N // CB
FR = NB // NS
EX = NB - FR * NS
TAIL = N - NB * CB
TROW = NB * CB
SEGT = S // NS


def _body(x_hbm, b_hbm, out_hbm, xbufs, ibufs, ibuft, acc, cnt, cbuf,
          shsum, shcnt, sem_g0, sem_g1, sem_b0, sem_b1, idxio, p2, obuf):
    c = lax.axis_index("c")
    s = lax.axis_index("s")
    lane = lax.iota(jnp.int32, L)
    zf = jnp.zeros((L,), jnp.float32)
    mask0 = lane == 0

    def zero_body(i, _):
        for j in range(NV):
            acc[i, pl.ds(j * L, L)] = zf
        return 0

    lax.fori_loop(0, S, zero_body, 0)
    for j in range(S // L):
        cnt[pl.ds(j * L, L)] = zf
        idxio[pl.ds(j * L, L)] = lane + j * L

    @pl.when(s == 0)
    def _():
        pltpu.sync_copy(acc, shsum)

    plsc.subcore_barrier()

    def reduce_block(ids_at, nrows, buf):
        def run_step(pos):
            sv = ids_at(pos)
            segv = sv.at[jnp.full((L,), pos & (L - 1), jnp.int32)].get(
                mode="promise_in_bounds")
            nrunv = jnp.zeros((L,), jnp.int32)
            for g in range(nrows // L):
                m = ids_at(g * L) == segv
                nrunv = nrunv + plsc.all_reduce_population_count(m)
            nrun = jnp.sum(jnp.where(mask0, nrunv, 0))

            def row4_body(i, vs):
                base = pos + i * 4
                out = []
                for j in range(NV):
                    a = (xbufs[buf, base, pl.ds(j * L, L)]
                         + xbufs[buf, base + 1, pl.ds(j * L, L)])
                    b = (xbufs[buf, base + 2, pl.ds(j * L, L)]
                         + xbufs[buf, base + 3, pl.ds(j * L, L)])
                    out.append(vs[j] + (a + b))
                return tuple(out)

            def row_body(i, vs):
                return tuple(
                    vs[j] + xbufs[buf, i, pl.ds(j * L, L)] for j in range(NV))

            n4 = nrun >> 2
            vs = lax.fori_loop(0, n4, row4_body, tuple(zf for _ in range(NV)))
            vs = lax.fori_loop(pos + (n4 << 2), pos + nrun, row_body, vs)
            for j in range(NV):
                plsc.addupdate_scatter(acc, [segv, lane + j * L], vs[j])
            plsc.addupdate_scatter(
                cnt, [segv], jnp.full((L,), nrun, jnp.int32).astype(jnp.float32),
                mask=mask0)
            return pos + nrun

        lax.while_loop(lambda p: p < nrows, run_step, jnp.int32(0))

    sem_g = (sem_g0, sem_g1)
    sem_b = (sem_b0, sem_b1)

    def gather(k, blk):
        return pltpu.async_copy(
            x_hbm.at[pl.ds(blk * CB, CB), pl.ds(c * DC, DC)],
            xbufs.at[k % 2], sem_g[k % 2])

    def bload(k, blk):
        last = None
        for q in range(3):
            last = pltpu.async_copy(
                b_hbm.at[pl.ds(blk * CB + q * 128, 128)],
                ibufs.at[k % 2, pl.ds(q * 128, 128)], sem_b[k % 2])
        return last

    def bwait(d):
        d.wait()
        d.wait()
        d.wait()

    gd = {0: gather(0, s)}
    bd = {0: bload(0, s)}
    for k in range(FR):
        blk = k * NS + s
        if k + 1 < FR:
            gd[k + 1] = gather(k + 1, blk + NS)
            bd[k + 1] = bload(k + 1, blk + NS)
        bwait(bd.pop(k))
        gd.pop(k).wait()

        def ids_at(pos, _k=k):
            off = pl.multiple_of((pos >> 4) * L, L)
            return ibufs[_k % 2, pl.ds(off, L)]

        reduce_block(ids_at, CB, k % 2)

    @pl.when(s < EX)
    def _():
        blk = FR * NS + s
        for q in range(3):
            pltpu.sync_copy(
                b_hbm.at[pl.ds(blk * CB + q * 128, 128)],
                ibufs.at[FR % 2, pl.ds(q * 128, 128)])
        pltpu.sync_copy(
            x_hbm.at[pl.ds(blk * CB, CB), pl.ds(c * DC, DC)],
            xbufs.at[FR % 2])

        def ids_at(pos):
            off = pl.multiple_of((pos >> 4) * L, L)
            return ibufs[FR % 2, pl.ds(off, L)]

        reduce_block(ids_at, CB, FR % 2)

    @pl.when(s == EX)
    def _():
        pltpu.sync_copy(b_hbm.at[pl.ds(TROW, TAIL)], ibuft)
        pltpu.sync_copy(
            x_hbm.at[pl.ds(TROW, TAIL), pl.ds(c * DC, DC)],
            xbufs.at[FR % 2, pl.ds(0, TAIL)])

        def ids_at(pos):
            off = pl.multiple_of((pos >> 4) * L, L)
            return ibuft[pl.ds(off, L)]

        reduce_block(ids_at, TAIL, FR % 2)

    pltpu.sync_copy(acc, shsum.at[idxio], add=True)
    pltpu.sync_copy(cnt, shcnt.at[s])
    plsc.subcore_barrier()

    pltpu.sync_copy(shsum.at[s * SEGT + (lane & (SEGT - 1))], p2)
    pltpu.sync_copy(shcnt, cbuf)

    cbase = (s // 2) * L
    cv = cbuf[0, pl.ds(cbase, L)]
    for t in range(1, NS):
        cv = cv + cbuf[t, pl.ds(cbase, L)]
    rv = 1.0 / jnp.maximum(cv, 1.0)

    def fin_body(i, _):
        lane_i = (s % 2) * SEGT + i
        rvec = rv.at[jnp.full((L,), lane_i, jnp.int32)].get(
            mode="promise_in_bounds")
        for j in range(NV):
            obuf[i, pl.ds(j * L, L)] = p2[i, pl.ds(j * L, L)] * rvec
        return 0

    lax.fori_loop(0, SEGT, fin_body, 0)
    pltpu.sync_copy(
        obuf, out_hbm.at[pl.ds(s * SEGT, SEGT), pl.ds(c * DC, DC)])


@functools.cache
def _build():
    mesh = plsc.VectorSubcoreMesh(
        core_axis_name="c", subcore_axis_name="s", num_cores=NC, num_subcores=NS
    )
    return pl.kernel(
        _body,
        out_type=jax.ShapeDtypeStruct((S, D), jnp.float32),
        mesh=mesh,
        compiler_params=pltpu.CompilerParams(needs_layout_passes=False),
        scratch_types=[
            pltpu.VMEM((2, CB, DC), jnp.float32),
            pltpu.VMEM((2, CB), jnp.int32),
            pltpu.VMEM((TAIL,), jnp.int32),
            pltpu.VMEM((S, DC), jnp.float32),
            pltpu.VMEM((S,), jnp.float32),
            pltpu.VMEM((NS, S), jnp.float32),
            pltpu.VMEM_SHARED((S, DC), jnp.float32),
            pltpu.VMEM_SHARED((NS, S), jnp.float32),
            pltpu.SemaphoreType.DMA,
            pltpu.SemaphoreType.DMA,
            pltpu.SemaphoreType.DMA,
            pltpu.SemaphoreType.DMA,
            pltpu.VMEM((S,), jnp.int32),
            pltpu.VMEM((L, DC), jnp.float32),
            pltpu.VMEM((SEGT, DC), jnp.float32),
        ],
    )


@jax.jit
def kernel(x, batch):
    return _build()(x, batch.astype(jnp.int32))


# speedup vs baseline: 1.0491x; 1.0491x over previous
import functools

import jax
import jax.numpy as jnp
from jax import lax
from jax.experimental import pallas as pl
from jax.experimental.pallas import tpu as pltpu
from jax.experimental.pallas import tpu_sc as plsc

N = 50000
D = 256
S = 128
NC = 2
NS = 16
L = 16
DC = D // NC
NV = DC // L
CB = 384
NB = N // CB
FR = NB // NS
EX = NB - FR * NS
TAIL = N - NB * CB
TROW = NB * CB
SEGT = S // NS


def _body(x_hbm, b_hbm, out_hbm, xbufs, ibufs, ibuft, acc, cnt, cbuf,
          shsum, shcnt, sem_g0, sem_g1, sem_b0, sem_b1, idxio, p2, obuf):
    c = lax.axis_index("c")
    s = lax.axis_index("s")
    lane = lax.iota(jnp.int32, L)
    zf = jnp.zeros((L,), jnp.float32)
    mask0 = lane == 0

    def zero_body(i, _):
        for j in range(NV):
            acc[i, pl.ds(j * L, L)] = zf
        return 0

    lax.fori_loop(0, S, zero_body, 0)
    for j in range(S // L):
        cnt[pl.ds(j * L, L)] = zf
        idxio[pl.ds(j * L, L)] = lane + j * L

    @pl.when(s == 0)
    def _():
        pltpu.sync_copy(acc, shsum)

    plsc.subcore_barrier()

    def reduce_block(ids_at, nrows, buf):
        def run_step(pos):
            sv = ids_at(pos)
            segv = sv.at[jnp.full((L,), pos & (L - 1), jnp.int32)].get(
                mode="promise_in_bounds")
            nrunv = jnp.zeros((L,), jnp.int32)
            for g in range(nrows // L):
                m = ids_at(g * L) == segv
                nrunv = nrunv + plsc.all_reduce_population_count(m)
            nrun = jnp.sum(jnp.where(mask0, nrunv, 0))

            def row_body(i, vs):
                return tuple(
                    vs[j] + xbufs[buf, i, pl.ds(j * L, L)] for j in range(NV))

            vs = lax.fori_loop(
                pos, pos + nrun, row_body, tuple(zf for _ in range(NV)))
            for j in range(NV):
                plsc.addupdate_scatter(acc, [segv, lane + j * L], vs[j])
            plsc.addupdate_scatter(
                cnt, [segv], jnp.full((L,), nrun, jnp.int32).astype(jnp.float32),
                mask=mask0)
            return pos + nrun

        lax.while_loop(lambda p: p < nrows, run_step, jnp.int32(0))

    sem_g = (sem_g0, sem_g1)
    sem_b = (sem_b0, sem_b1)

    H = CB // 2

    def gather(k, blk):
        pltpu.async_copy(
            x_hbm.at[pl.ds(blk * CB, H), pl.ds(c * DC, DC)],
            xbufs.at[k % 2, pl.ds(0, H)], sem_g[k % 2])
        return pltpu.async_copy(
            x_hbm.at[pl.ds(blk * CB + H, H), pl.ds(c * DC, DC)],
            xbufs.at[k % 2, pl.ds(H, H)], sem_g[k % 2])

    def bload(k, blk):
        last = None
        for q in range(3):
            last = pltpu.async_copy(
                b_hbm.at[pl.ds(blk * CB + q * 128, 128)],
                ibufs.at[k % 2, pl.ds(q * 128, 128)], sem_b[k % 2])
        return last

    def bwait(d):
        d.wait()
        d.wait()
        d.wait()

    gd = {0: gather(0, s)}
    bd = {0: bload(0, s)}
    for k in range(FR):
        blk = k * NS + s
        if k + 1 < FR:
            gd[k + 1] = gather(k + 1, blk + NS)
            bd[k + 1] = bload(k + 1, blk + NS)
        bwait(bd.pop(k))
        g = gd.pop(k)
        g.wait()
        g.wait()

        def ids_at(pos, _k=k):
            off = pl.multiple_of((pos >> 4) * L, L)
            return ibufs[_k % 2, pl.ds(off, L)]

        reduce_block(ids_at, CB, k % 2)

    @pl.when(s < EX)
    def _():
        blk = FR * NS + s
        for q in range(3):
            pltpu.sync_copy(
                b_hbm.at[pl.ds(blk * CB + q * 128, 128)],
                ibufs.at[FR % 2, pl.ds(q * 128, 128)])
        pltpu.sync_copy(
            x_hbm.at[pl.ds(blk * CB, CB), pl.ds(c * DC, DC)],
            xbufs.at[FR % 2])

        def ids_at(pos):
            off = pl.multiple_of((pos >> 4) * L, L)
            return ibufs[FR % 2, pl.ds(off, L)]

        reduce_block(ids_at, CB, FR % 2)

    @pl.when(s == EX)
    def _():
        pltpu.sync_copy(b_hbm.at[pl.ds(TROW, TAIL)], ibuft)
        pltpu.sync_copy(
            x_hbm.at[pl.ds(TROW, TAIL), pl.ds(c * DC, DC)],
            xbufs.at[FR % 2, pl.ds(0, TAIL)])

        def ids_at(pos):
            off = pl.multiple_of((pos >> 4) * L, L)
            return ibuft[pl.ds(off, L)]

        reduce_block(ids_at, TAIL, FR % 2)

    pltpu.sync_copy(acc, shsum.at[idxio], add=True)
    pltpu.sync_copy(cnt, shcnt.at[s])
    plsc.subcore_barrier()

    pltpu.sync_copy(shsum.at[s * SEGT + (lane & (SEGT - 1))], p2)
    pltpu.sync_copy(shcnt, cbuf)

    cbase = (s // 2) * L
    cv = cbuf[0, pl.ds(cbase, L)]
    for t in range(1, NS):
        cv = cv + cbuf[t, pl.ds(cbase, L)]
    rv = 1.0 / jnp.maximum(cv, 1.0)

    def fin_body(i, _):
        lane_i = (s % 2) * SEGT + i
        rvec = rv.at[jnp.full((L,), lane_i, jnp.int32)].get(
            mode="promise_in_bounds")
        for j in range(NV):
            obuf[i, pl.ds(j * L, L)] = p2[i, pl.ds(j * L, L)] * rvec
        return 0

    lax.fori_loop(0, SEGT, fin_body, 0)
    pltpu.sync_copy(
        obuf, out_hbm.at[pl.ds(s * SEGT, SEGT), pl.ds(c * DC, DC)])


@functools.cache
def _build():
    mesh = plsc.VectorSubcoreMesh(
        core_axis_name="c", subcore_axis_name="s", num_cores=NC, num_subcores=NS
    )
    return pl.kernel(
        _body,
        out_type=jax.ShapeDtypeStruct((S, D), jnp.float32),
        mesh=mesh,
        compiler_params=pltpu.CompilerParams(needs_layout_passes=False),
        scratch_types=[
            pltpu.VMEM((2, CB, DC), jnp.float32),
            pltpu.VMEM((2, CB), jnp.int32),
            pltpu.VMEM((TAIL,), jnp.int32),
            pltpu.VMEM((S, DC), jnp.float32),
            pltpu.VMEM((S,), jnp.float32),
            pltpu.VMEM((NS, S), jnp.float32),
            pltpu.VMEM_SHARED((S, DC), jnp.float32),
            pltpu.VMEM_SHARED((NS, S), jnp.float32),
            pltpu.SemaphoreType.DMA,
            pltpu.SemaphoreType.DMA,
            pltpu.SemaphoreType.DMA,
            pltpu.SemaphoreType.DMA,
            pltpu.VMEM((S,), jnp.int32),
            pltpu.VMEM((L, DC), jnp.float32),
            pltpu.VMEM((SEGT, DC), jnp.float32),
        ],
    )


@jax.jit
def kernel(x, batch):
    return _build()(x, batch.astype(jnp.int32))


# speedup vs baseline: 1.0572x; 1.0078x over previous
import functools

import jax
import jax.numpy as jnp
from jax import lax
from jax.experimental import pallas as pl
from jax.experimental.pallas import tpu as pltpu
from jax.experimental.pallas import tpu_sc as plsc

N = 50000
D = 256
S = 128
NC = 2
NS = 16
L = 16
DC = D // NC
NV = DC // L
CB = 384
NB = N // CB
FR = NB // NS
EX = NB - FR * NS
TAIL = N - NB * CB
TROW = NB * CB
SEGT = S // NS


def _body(x_hbm, b_hbm, out_hbm, xbufs, ibufs, ibuft, acc, cnt, cbuf,
          shsum, shcnt, sem_g0, sem_g1, sem_b0, sem_b1, idxio, p2, obuf):
    c = lax.axis_index("c")
    s = lax.axis_index("s")
    lane = lax.iota(jnp.int32, L)
    zf = jnp.zeros((L,), jnp.float32)
    mask0 = lane == 0

    def zero_body(i, _):
        for j in range(NV):
            acc[i, pl.ds(j * L, L)] = zf
        return 0

    lax.fori_loop(0, S, zero_body, 0)
    for j in range(S // L):
        cnt[pl.ds(j * L, L)] = zf
        idxio[pl.ds(j * L, L)] = lane + j * L

    @pl.when(s == 0)
    def _():
        pltpu.sync_copy(acc, shsum)

    plsc.subcore_barrier()

    def reduce_block(ids_at, nrows, buf):
        def run_step(pos):
            sv = ids_at(pos)
            segv = sv.at[jnp.full((L,), pos & (L - 1), jnp.int32)].get(
                mode="promise_in_bounds")
            nrunv = jnp.zeros((L,), jnp.int32)
            for g in range(nrows // L):
                m = ids_at(g * L) == segv
                nrunv = nrunv + plsc.all_reduce_population_count(m)
            nrun = jnp.sum(jnp.where(mask0, nrunv, 0))

            def row_body(i, vs):
                return tuple(
                    vs[j] + xbufs[buf, i, pl.ds(j * L, L)] for j in range(NV))

            vs = lax.fori_loop(
                pos, pos + nrun, row_body, tuple(zf for _ in range(NV)))
            for j in range(NV):
                plsc.addupdate_scatter(acc, [segv, lane + j * L], vs[j])
            plsc.addupdate_scatter(
                cnt, [segv], jnp.full((L,), nrun, jnp.int32).astype(jnp.float32),
                mask=mask0)
            return pos + nrun

        lax.while_loop(lambda p: p < nrows, run_step, jnp.int32(0))

    sem_g = (sem_g0, sem_g1)
    sem_b = (sem_b0, sem_b1)

    def gather(k, blk):
        return pltpu.async_copy(
            x_hbm.at[pl.ds(blk * CB, CB), pl.ds(c * DC, DC)],
            xbufs.at[k % 2], sem_g[k % 2])

    def bload(k, blk):
        last = None
        for q in range(3):
            last = pltpu.async_copy(
                b_hbm.at[pl.ds(blk * CB + q * 128, 128)],
                ibufs.at[k % 2, pl.ds(q * 128, 128)], sem_b[k % 2])
        return last

    def bwait(d):
        d.wait()
        d.wait()
        d.wait()

    gd = {0: gather(0, s)}
    bd = {0: bload(0, s)}
    for k in range(FR):
        blk = k * NS + s
        if k + 1 < FR:
            gd[k + 1] = gather(k + 1, blk + NS)
            bd[k + 1] = bload(k + 1, blk + NS)
        bwait(bd.pop(k))
        gd.pop(k).wait()

        def ids_at(pos, _k=k):
            off = pl.multiple_of((pos >> 4) * L, L)
            return ibufs[_k % 2, pl.ds(off, L)]

        reduce_block(ids_at, CB, k % 2)

    @pl.when(s < EX)
    def _():
        blk = FR * NS + s
        for q in range(3):
            pltpu.sync_copy(
                b_hbm.at[pl.ds(blk * CB + q * 128, 128)],
                ibufs.at[FR % 2, pl.ds(q * 128, 128)])
        pltpu.sync_copy(
            x_hbm.at[pl.ds(blk * CB, CB), pl.ds(c * DC, DC)],
            xbufs.at[FR % 2])

        def ids_at(pos):
            off = pl.multiple_of((pos >> 4) * L, L)
            return ibufs[FR % 2, pl.ds(off, L)]

        reduce_block(ids_at, CB, FR % 2)

    @pl.when(s == EX)
    def _():
        pltpu.sync_copy(b_hbm.at[pl.ds(TROW, TAIL)], ibuft)
        pltpu.sync_copy(
            x_hbm.at[pl.ds(TROW, TAIL), pl.ds(c * DC, DC)],
            xbufs.at[FR % 2, pl.ds(0, TAIL)])

        def ids_at(pos):
            off = pl.multiple_of((pos >> 4) * L, L)
            return ibuft[pl.ds(off, L)]

        reduce_block(ids_at, TAIL, FR % 2)

    pltpu.sync_copy(acc, shsum.at[idxio], add=True)
    pltpu.sync_copy(cnt, shcnt.at[s])
    plsc.subcore_barrier()

    pltpu.sync_copy(shsum.at[s * SEGT + (lane & (SEGT - 1))], p2)
    pltpu.sync_copy(shcnt, cbuf)

    cbase = (s // 2) * L
    cv = cbuf[0, pl.ds(cbase, L)]
    for t in range(1, NS):
        cv = cv + cbuf[t, pl.ds(cbase, L)]
    rv = 1.0 / jnp.maximum(cv, 1.0)

    def fin_body(i, _):
        lane_i = (s % 2) * SEGT + i
        rvec = rv.at[jnp.full((L,), lane_i, jnp.int32)].get(
            mode="promise_in_bounds")
        for j in range(NV):
            obuf[i, pl.ds(j * L, L)] = p2[i, pl.ds(j * L, L)] * rvec
        return 0

    lax.fori_loop(0, SEGT, fin_body, 0)
    pltpu.sync_copy(
        obuf, out_hbm.at[pl.ds(s * SEGT, SEGT), pl.ds(c * DC, DC)])


@functools.cache
def _build():
    mesh = plsc.VectorSubcoreMesh(
        core_axis_name="c", subcore_axis_name="s", num_cores=NC, num_subcores=NS
    )
    return pl.kernel(
        _body,
        out_type=jax.ShapeDtypeStruct((S, D), jnp.float32),
        mesh=mesh,
        compiler_params=pltpu.CompilerParams(
            needs_layout_passes=False, disable_bounds_checks=True),
        scratch_types=[
            pltpu.VMEM((2, CB, DC), jnp.float32),
            pltpu.VMEM((2, CB), jnp.int32),
            pltpu.VMEM((TAIL,), jnp.int32),
            pltpu.VMEM((S, DC), jnp.float32),
            pltpu.VMEM((S,), jnp.float32),
            pltpu.VMEM((NS, S), jnp.float32),
            pltpu.VMEM_SHARED((S, DC), jnp.float32),
            pltpu.VMEM_SHARED((NS, S), jnp.float32),
            pltpu.SemaphoreType.DMA,
            pltpu.SemaphoreType.DMA,
            pltpu.SemaphoreType.DMA,
            pltpu.SemaphoreType.DMA,
            pltpu.VMEM((S,), jnp.int32),
            pltpu.VMEM((L, DC), jnp.float32),
            pltpu.VMEM((SEGT, DC), jnp.float32),
        ],
    )


@jax.jit
def kernel(x, batch):
    return _build()(x, batch.astype(jnp.int32))


# speedup vs baseline: 1.0579x; 1.0007x over previous
import functools

import jax
import jax.numpy as jnp
from jax import lax
from jax.experimental import pallas as pl
from jax.experimental.pallas import tpu as pltpu
from jax.experimental.pallas import tpu_sc as plsc

N = 50000
D = 256
S = 128
NC = 2
NS = 16
L = 16
DC = D // NC
NV = DC // L
CB = 384
NB = N // CB
FR = NB // NS
EX = NB - FR * NS
TAIL = N - NB * CB
TROW = NB * CB
SEGT = S // NS


def _body(x_hbm, b_hbm, out_hbm, xbufs, ibufs, ibuft, acc, cnt, cbuf,
          shsum, shcnt, sem_g0, sem_g1, sem_b0, sem_b1, idxio, p2, obuf):
    c = lax.axis_index("c")
    s = lax.axis_index("s")
    lane = lax.iota(jnp.int32, L)
    zf = jnp.zeros((L,), jnp.float32)
    mask0 = lane == 0

    def zero_body(i, _):
        for j in range(NV):
            acc[i, pl.ds(j * L, L)] = zf
        return 0

    lax.fori_loop(0, S, zero_body, 0)
    for j in range(S // L):
        cnt[pl.ds(j * L, L)] = zf
        idxio[pl.ds(j * L, L)] = lane + j * L

    @pl.when(s == 0)
    def _():
        pltpu.sync_copy(acc, shsum)

    plsc.subcore_barrier()

    def reduce_block(ids_at, nrows, buf):
        def run_step(pos):
            sv = ids_at(pos)
            segv = sv.at[jnp.full((L,), pos & (L - 1), jnp.int32)].get(
                mode="promise_in_bounds")
            nrunv = jnp.zeros((L,), jnp.int32)
            for g in range(nrows // L):
                m = ids_at(g * L) == segv
                nrunv = nrunv + plsc.all_reduce_population_count(m)
            nrun = jnp.sum(jnp.where(mask0, nrunv, 0))

            def row_body(i, vs):
                return tuple(
                    vs[j] + xbufs[buf, i, pl.ds(j * L, L)] for j in range(NV))

            vs = lax.fori_loop(
                pos, pos + nrun, row_body, tuple(zf for _ in range(NV)))
            for j in range(NV):
                plsc.addupdate_scatter(acc, [segv, lane + j * L], vs[j])
            plsc.addupdate_scatter(
                cnt, [segv], jnp.full((L,), nrun, jnp.int32).astype(jnp.float32),
                mask=mask0)
            return pos + nrun

        lax.while_loop(lambda p: p < nrows, run_step, jnp.int32(0))

    sem_g = (sem_g0, sem_g1)
    sem_b = (sem_b0, sem_b1)

    def gather(k, blk):
        return pltpu.async_copy(
            x_hbm.at[pl.ds(blk * CB, CB), pl.ds(c * DC, DC)],
            xbufs.at[k % 2], sem_g[k % 2])

    def bload(k, blk):
        last = None
        for q in range(3):
            last = pltpu.async_copy(
                b_hbm.at[pl.ds(blk * CB + q * 128, 128)],
                ibufs.at[k % 2, pl.ds(q * 128, 128)], sem_b[k % 2])
        return last

    def bwait(d):
        d.wait()
        d.wait()
        d.wait()

    gd = {0: gather(0, s)}
    bd = {0: bload(0, s)}
    for k in range(FR):
        blk = k * NS + s
        if k + 1 < FR:
            gd[k + 1] = gather(k + 1, blk + NS)
            bd[k + 1] = bload(k + 1, blk + NS)
        bwait(bd.pop(k))
        gd.pop(k).wait()

        def ids_at(pos, _k=k):
            off = pl.multiple_of((pos >> 4) * L, L)
            return ibufs[_k % 2, pl.ds(off, L)]

        reduce_block(ids_at, CB, k % 2)

    @pl.when(s < EX)
    def _():
        blk = FR * NS + s
        for q in range(3):
            pltpu.sync_copy(
                b_hbm.at[pl.ds(blk * CB + q * 128, 128)],
                ibufs.at[FR % 2, pl.ds(q * 128, 128)])
        pltpu.sync_copy(
            x_hbm.at[pl.ds(blk * CB, CB), pl.ds(c * DC, DC)],
            xbufs.at[FR % 2])

        def ids_at(pos):
            off = pl.multiple_of((pos >> 4) * L, L)
            return ibufs[FR % 2, pl.ds(off, L)]

        reduce_block(ids_at, CB, FR % 2)

    @pl.when(s == EX)
    def _():
        pltpu.sync_copy(b_hbm.at[pl.ds(TROW, TAIL)], ibuft)
        pltpu.sync_copy(
            x_hbm.at[pl.ds(TROW, TAIL), pl.ds(c * DC, DC)],
            xbufs.at[FR % 2, pl.ds(0, TAIL)])

        def ids_at(pos):
            off = pl.multiple_of((pos >> 4) * L, L)
            return ibuft[pl.ds(off, L)]

        reduce_block(ids_at, TAIL, FR % 2)

    pltpu.sync_copy(acc, shsum.at[idxio], add=True)
    pltpu.sync_copy(cnt, shcnt.at[s])
    plsc.subcore_barrier()

    pltpu.sync_copy(shsum.at[s * SEGT + (lane & (SEGT - 1))], p2)
    pltpu.sync_copy(shcnt, cbuf)

    cbase = (s // 2) * L
    cv = cbuf[0, pl.ds(cbase, L)]
    for t in range(1, NS):
        cv = cv + cbuf[t, pl.ds(cbase, L)]
    rv = 1.0 / jnp.maximum(cv, 1.0)

    def fin_body(i, _):
        lane_i = (s % 2) * SEGT + i
        rvec = rv.at[jnp.full((L,), lane_i, jnp.int32)].get(
            mode="promise_in_bounds")
        for j in range(NV):
            obuf[i, pl.ds(j * L, L)] = p2[i, pl.ds(j * L, L)] * rvec
        return 0

    lax.fori_loop(0, SEGT, fin_body, 0)
    pltpu.sync_copy(
        obuf, out_hbm.at[pl.ds(s * SEGT, SEGT), pl.ds(c * DC, DC)])


@functools.cache
def _build():
    mesh = plsc.VectorSubcoreMesh(
        core_axis_name="c", subcore_axis_name="s", num_cores=NC, num_subcores=NS
    )
    return pl.kernel(
        _body,
        out_type=jax.ShapeDtypeStruct((S, D), jnp.float32),
        mesh=mesh,
        compiler_params=pltpu.CompilerParams(needs_layout_passes=False),
        scratch_types=[
            pltpu.VMEM((2, CB, DC), jnp.float32),
            pltpu.VMEM((2, CB), jnp.int32),
            pltpu.VMEM((TAIL,), jnp.int32),
            pltpu.VMEM((S, DC), jnp.float32),
            pltpu.VMEM((S,), jnp.float32),
            pltpu.VMEM((NS, S), jnp.float32),
            pltpu.VMEM_SHARED((S, DC), jnp.float32),
            pltpu.VMEM_SHARED((NS, S), jnp.float32),
            pltpu.SemaphoreType.DMA,
            pltpu.SemaphoreType.DMA,
            pltpu.SemaphoreType.DMA,
            pltpu.SemaphoreType.DMA,
            pltpu.VMEM((S,), jnp.int32),
            pltpu.VMEM((L, DC), jnp.float32),
            pltpu.VMEM((SEGT, DC), jnp.float32),
        ],
    )


@jax.jit
def kernel(x, batch):
    return _build()(x, batch.astype(jnp.int32))
